# baseline (speedup 1.0000x reference)
import dataclasses
import functools

import jax
import jax.numpy as jnp
from jax import lax
from jax.experimental import pallas as pl
from jax.experimental.pallas import tpu as pltpu
from jax.experimental.pallas import tpu_sc as plsc

N = 10000
NPAD = 10240
E = 320000
C = 128

NC = 2
NS = 16
NW = NC * NS
EW = E // NW
K = 80
NCHUNK = EW // K
KW = 2000
NACC = NPAD
RSUB = NACC // NS
NG = C // 16


def _prep_body(x_ref, w_ref, as_ref, ad_ref, xw_ref, asrc_ref, adst_ref,
               negm_ref, mscr):
    i = pl.program_id(0)
    xwb = jnp.dot(x_ref[...], w_ref[...], preferred_element_type=jnp.float32)
    xw_ref[...] = xwb
    a_s = jnp.sum(xwb * as_ref[...], axis=1, keepdims=True)
    a_d = jnp.sum(xwb * ad_ref[...], axis=1, keepdims=True)
    asrc_ref[...] = a_s
    adst_ref[...] = a_d
    ms = jnp.max(a_s)
    md = jnp.max(a_d)

    @pl.when(i == 0)
    def _():
        mscr[0] = ms
        mscr[1] = md

    @pl.when(i > 0)
    def _():
        mscr[0] = jnp.maximum(mscr[0], ms)
        mscr[1] = jnp.maximum(mscr[1], md)

    @pl.when(i == NPAD // 128 - 1)
    def _():
        negm_ref[0, 0] = -jnp.maximum(mscr[0] + mscr[1], 0.0)


_prep_call = pl.pallas_call(
    _prep_body,
    grid=(NPAD // 128,),
    in_specs=[
        pl.BlockSpec((128, C), lambda i: (i, 0)),
        pl.BlockSpec((C, C), lambda i: (0, 0)),
        pl.BlockSpec((1, C), lambda i: (0, 0)),
        pl.BlockSpec((1, C), lambda i: (0, 0)),
    ],
    out_specs=[
        pl.BlockSpec((128, C), lambda i: (i, 0)),
        pl.BlockSpec((128, 1), lambda i: (i, 0)),
        pl.BlockSpec((128, 1), lambda i: (i, 0)),
        pl.BlockSpec(memory_space=pltpu.SMEM),
    ],
    out_shape=[
        jax.ShapeDtypeStruct((NPAD, C), jnp.float32),
        jax.ShapeDtypeStruct((NPAD, 1), jnp.float32),
        jax.ShapeDtypeStruct((NPAD, 1), jnp.float32),
        jax.ShapeDtypeStruct((1, 1), jnp.float32),
    ],
    scratch_shapes=[pltpu.SMEM((2,), jnp.float32)],
)


_vector_mesh = plsc.VectorSubcoreMesh(core_axis_name="c", subcore_axis_name="s")

_sc_params = pltpu.CompilerParams()
if "needs_layout_passes" in pltpu.CompilerParams.__dataclass_fields__:
    _sc_params = dataclasses.replace(_sc_params, needs_layout_passes=False)


@functools.partial(
    pl.kernel,
    compiler_params=_sc_params,
    out_type=(
        jax.ShapeDtypeStruct((E,), jnp.float32),
        jax.ShapeDtypeStruct((NC, NPAD), jnp.float32),
    ),
    mesh=_vector_mesh,
    scratch_types=[
        pltpu.VMEM((NPAD,), jnp.float32),
        pltpu.VMEM((NPAD,), jnp.float32),
        pltpu.VMEM((16,), jnp.float32),
        pltpu.VMEM((KW,), jnp.int32),
        pltpu.VMEM((KW,), jnp.int32),
        pltpu.VMEM((KW,), jnp.float32),
        pltpu.VMEM((NPAD,), jnp.float32),
        pltpu.VMEM((RSUB,), jnp.float32),
        pltpu.VMEM((RSUB,), jnp.float32),
        pltpu.VMEM_SHARED((NS, NPAD), jnp.float32),
    ],
)
def _edge_w(asrc_hbm, adst_hbm, negm_hbm, src_hbm, dst_hbm, w_hbm, s_hbm,
            asrc_v, adst_v, negm_v, src_v, dst_v, w_v, sacc_v, sred_v, stmp_v,
            s_st):
    cid = lax.axis_index("c")
    sid = lax.axis_index("s")
    wid = sid * NC + cid

    pltpu.sync_copy(asrc_hbm, asrc_v)
    pltpu.sync_copy(adst_hbm, adst_v)
    pltpu.sync_copy(negm_hbm, negm_v)
    negm16 = negm_v[...]
    zf = jnp.zeros((16,), jnp.float32)
    iota16 = lax.iota(jnp.int32, 16)

    @pl.loop(0, NPAD, step=16)
    def _(t):
        sacc_v[pl.ds(t, 16)] = zf

    ebase = wid * EW

    @pl.loop(0, EW // KW)
    def _(c):
        base = ebase + c * KW
        pltpu.sync_copy(src_hbm.at[pl.ds(base, KW)], src_v)
        pltpu.sync_copy(dst_hbm.at[pl.ds(base, KW)], dst_v)

        @pl.loop(0, KW, step=16)
        def _(g):
            s16 = src_v[pl.ds(g, 16)]
            d16 = dst_v[pl.ds(g, 16)]
            av = plsc.load_gather(asrc_v, [s16])
            bv = plsc.load_gather(adst_v, [d16])
            z = av + bv
            alpha = jnp.maximum(z, 0.2 * z)
            w = jnp.exp(alpha + negm16)
            w_v[pl.ds(g, 16)] = w
            for j in range(16):
                plsc.addupdate_scatter(sacc_v, [d16], w, mask=iota16 == j)

        pltpu.sync_copy(w_v, w_hbm.at[pl.ds(base, KW)])

    pltpu.sync_copy(sacc_v, s_st.at[sid])
    plsc.subcore_barrier()

    row0 = sid * RSUB

    pltpu.sync_copy(s_st.at[0, pl.ds(row0, RSUB)], sred_v)
    for k in range(1, NS):
        pltpu.sync_copy(s_st.at[k, pl.ds(row0, RSUB)], stmp_v)

        @pl.loop(0, RSUB, step=16)
        def _(t):
            sred_v[pl.ds(t, 16)] = sred_v[pl.ds(t, 16)] + stmp_v[pl.ds(t, 16)]

    pltpu.sync_copy(sred_v, s_hbm.at[cid, pl.ds(row0, RSUB)])


NBUF = 3
NIB = 4


@functools.partial(
    pl.kernel,
    compiler_params=_sc_params,
    out_type=jax.ShapeDtypeStruct((NC, NACC, C), jnp.float32),
    mesh=_vector_mesh,
    scratch_types=[
        [pltpu.VMEM((K,), jnp.int32)] * NIB,
        [pltpu.VMEM((K,), jnp.int32)] * NIB,
        [pltpu.VMEM((K,), jnp.float32)] * NIB,
        [pltpu.VMEM((K, C), jnp.float32)] * NBUF,
        pltpu.VMEM_SHARED((NACC, C), jnp.float32),
        [pltpu.SemaphoreType.DMA] * NBUF,
        [pltpu.SemaphoreType.DMA] * NBUF,
        [pltpu.SemaphoreType.DMA] * NIB,
    ],
)
def _gat_scatter(xw_hbm, w_hbm, src_hbm, dst_hbm, o_part,
                 src_v, dst_v, w_v, rows_v, out_sh, gsem, ssem, isem):
    cid = lax.axis_index("c")
    sid = lax.axis_index("s")
    wid = sid * NC + cid

    zf = jnp.zeros((16,), jnp.float32)

    @pl.loop(0, K)
    def _(r):
        for g in range(NG):
            rows_v[0][r, pl.ds(g * 16, 16)] = zf

    row0 = sid * RSUB
    for j in range(RSUB // K):
        pltpu.sync_copy(rows_v[0], out_sh.at[pl.ds(row0 + j * K, K)])

    plsc.subcore_barrier()

    ebase = wid * EW
    zi = jnp.zeros((16,), jnp.int32)

    def load_idx_sync(c, i4):
        base = ebase + c * K
        pltpu.sync_copy(src_hbm.at[pl.ds(base, K)], src_v[i4])
        pltpu.sync_copy(dst_hbm.at[pl.ds(base, K)], dst_v[i4])
        pltpu.sync_copy(w_hbm.at[pl.ds(base, K)], w_v[i4])

    def load_idx_async(c, i4):
        base = ebase + c * K
        pltpu.async_copy(src_hbm.at[pl.ds(base, K)], src_v[i4], isem[i4])
        pltpu.async_copy(dst_hbm.at[pl.ds(base, K)], dst_v[i4], isem[i4])
        pltpu.async_copy(w_hbm.at[pl.ds(base, K)], w_v[i4], isem[i4])

    def wait_idx(c, i4):
        base = ebase + c * K
        pltpu.make_async_copy(src_hbm.at[pl.ds(base, K)], src_v[i4], isem[i4]).wait()
        pltpu.make_async_copy(dst_hbm.at[pl.ds(base, K)], dst_v[i4], isem[i4]).wait()
        pltpu.make_async_copy(w_hbm.at[pl.ds(base, K)], w_v[i4], isem[i4]).wait()

    def start_gather(i4, b):
        pltpu.async_copy(xw_hbm.at[src_v[i4]], rows_v[b], gsem[b])

    def wait_gather(i4, b):
        pltpu.make_async_copy(xw_hbm.at[src_v[i4]], rows_v[b], gsem[b]).wait()

    def scale_and_scatter(i4, b):
        @pl.loop(0, K, unroll=8)
        def _(r):
            wv = plsc.load_gather(w_v[i4], [zi + r])
            for g2 in range(NG):
                rows_v[b][r, pl.ds(g2 * 16, 16)] = (
                    rows_v[b][r, pl.ds(g2 * 16, 16)] * wv)

        pltpu.async_copy(rows_v[b], out_sh.at[dst_v[i4]], ssem[b], add=True)

    def wait_scatter(i4, b):
        pltpu.make_async_copy(rows_v[b], out_sh.at[dst_v[i4]], ssem[b]).wait()

    load_idx_sync(0, 0)
    start_gather(0, 0)
    load_idx_sync(1, 1)
    start_gather(1, 1)
    load_idx_async(2, 2)
    wait_gather(0, 0)
    scale_and_scatter(0, 0)
    wait_idx(2, 2)
    start_gather(2, 2)
    load_idx_async(3, 3)
    wait_gather(1, 1)
    scale_and_scatter(1, 1)

    @pl.loop(0, (NCHUNK - 5) // 12)
    def _(i):
        for u in range(12):
            c = 12 * i + 2 + u
            b = (2 + u) % NBUF
            bn = (b + 1) % NBUF
            i4 = (2 + u) % NIB
            i4n = (i4 + 1) % NIB
            i4nn = (i4 + 2) % NIB
            wait_scatter(i4nn, bn)
            wait_idx(c + 1, i4n)
            start_gather(i4n, bn)
            base2 = ebase + (c + 2) * K
            pltpu.async_copy(src_hbm.at[pl.ds(base2, K)], src_v[i4nn], isem[i4nn])
            pltpu.async_copy(dst_hbm.at[pl.ds(base2, K)], dst_v[i4nn], isem[i4nn])
            pltpu.async_copy(w_hbm.at[pl.ds(base2, K)], w_v[i4nn], isem[i4nn])
            wait_gather(i4, b)
            scale_and_scatter(i4, b)

    for c in range(NCHUNK - 3, NCHUNK):
        b = c % NBUF
        bn = (b + 1) % NBUF
        i4 = c % NIB
        i4n = (i4 + 1) % NIB
        i4nn = (i4 + 2) % NIB
        wait_scatter(i4nn, bn)
        if c + 1 < NCHUNK:
            wait_idx(c + 1, i4n)
            start_gather(i4n, bn)
        if c + 2 < NCHUNK:
            load_idx_async(c + 2, i4nn)
        wait_gather(i4, b)
        scale_and_scatter(i4, b)

    wait_scatter((NCHUNK - 2) % NIB, (NCHUNK - 2) % NBUF)
    wait_scatter((NCHUNK - 1) % NIB, (NCHUNK - 1) % NBUF)

    plsc.subcore_barrier()

    pltpu.sync_copy(out_sh.at[pl.ds(row0, RSUB)], o_part.at[cid, pl.ds(row0, RSUB)])


BLKC = 1000


def _final_body(o_ref, s_ref, b_ref, fw_ref, fb_ref, y_ref):
    o = o_ref[0] + o_ref[1]
    s = s_ref[0] + s_ref[1]
    h = o / (s + 1e-16) + b_ref[...]
    h = jnp.maximum(h, 0.0)
    y = jnp.sum(h * fw_ref[...], axis=1, keepdims=True) + fb_ref[0]
    y_ref[...] = jax.nn.sigmoid(y)


_final_call = pl.pallas_call(
    _final_body,
    grid=(N // BLKC,),
    in_specs=[
        pl.BlockSpec((NC, BLKC, C), lambda i: (0, i, 0)),
        pl.BlockSpec((NC, BLKC, 1), lambda i: (0, i, 0)),
        pl.BlockSpec((1, C), lambda i: (0, 0)),
        pl.BlockSpec((1, C), lambda i: (0, 0)),
        pl.BlockSpec(memory_space=pltpu.SMEM),
    ],
    out_specs=pl.BlockSpec((BLKC, 1), lambda i: (i, 0)),
    out_shape=jax.ShapeDtypeStruct((N, 1), jnp.float32),
)


def kernel(x, edge_index, W, att_src, att_dst, bias, fc_W, fc_b):
    src = edge_index[0]
    dst = edge_index[1]
    x_pad = jnp.pad(x, ((0, NPAD - N), (0, 0)))
    att_s = att_src.reshape(1, C)
    att_d = att_dst.reshape(1, C)
    xw, asrc2, adst2, negm = _prep_call(x_pad, W, att_s, att_d)
    asrc = asrc2.reshape(NPAD)
    adst = adst2.reshape(NPAD)
    negm16 = jnp.broadcast_to(negm.reshape(1), (16,))
    w_e, s_part = _edge_w(asrc, adst, negm16, src, dst)
    o_part = _gat_scatter(xw, w_e, src, dst)
    s_part3 = s_part.reshape(NC, NPAD, 1)
    return _final_call(o_part, s_part3, bias.reshape(1, C), fc_W.reshape(1, C),
                       fc_b)


# speedup vs baseline: 1.0218x; 1.0218x over previous
import dataclasses
import functools

import jax
import jax.numpy as jnp
from jax import lax
from jax.experimental import pallas as pl
from jax.experimental.pallas import tpu as pltpu
from jax.experimental.pallas import tpu_sc as plsc

N = 10000
NPAD = 10240
E = 320000
C = 128

NC = 2
NS = 16
NW = NC * NS
EW = E // NW
K = 80
NCHUNK = EW // K
KW = 2000
NACC = NPAD
RSUB = NACC // NS
NG = C // 16


def _prep_body(x_ref, w_ref, as_ref, ad_ref, xw_ref, asrc_ref, adst_ref,
               negm_ref, mscr):
    i = pl.program_id(0)
    xwb = jnp.dot(x_ref[...], w_ref[...], preferred_element_type=jnp.float32)
    xw_ref[...] = xwb
    a_s = jnp.sum(xwb * as_ref[...], axis=1, keepdims=True)
    a_d = jnp.sum(xwb * ad_ref[...], axis=1, keepdims=True)
    asrc_ref[...] = a_s
    adst_ref[...] = a_d
    ms = jnp.max(a_s)
    md = jnp.max(a_d)

    @pl.when(i == 0)
    def _():
        mscr[0] = ms
        mscr[1] = md

    @pl.when(i > 0)
    def _():
        mscr[0] = jnp.maximum(mscr[0], ms)
        mscr[1] = jnp.maximum(mscr[1], md)

    @pl.when(i == NPAD // 128 - 1)
    def _():
        negm_ref[0, 0] = -jnp.maximum(mscr[0] + mscr[1], 0.0)


_prep_call = pl.pallas_call(
    _prep_body,
    grid=(NPAD // 128,),
    in_specs=[
        pl.BlockSpec((128, C), lambda i: (i, 0)),
        pl.BlockSpec((C, C), lambda i: (0, 0)),
        pl.BlockSpec((1, C), lambda i: (0, 0)),
        pl.BlockSpec((1, C), lambda i: (0, 0)),
    ],
    out_specs=[
        pl.BlockSpec((128, C), lambda i: (i, 0)),
        pl.BlockSpec((128, 1), lambda i: (i, 0)),
        pl.BlockSpec((128, 1), lambda i: (i, 0)),
        pl.BlockSpec(memory_space=pltpu.SMEM),
    ],
    out_shape=[
        jax.ShapeDtypeStruct((NPAD, C), jnp.float32),
        jax.ShapeDtypeStruct((NPAD, 1), jnp.float32),
        jax.ShapeDtypeStruct((NPAD, 1), jnp.float32),
        jax.ShapeDtypeStruct((1, 1), jnp.float32),
    ],
    scratch_shapes=[pltpu.SMEM((2,), jnp.float32)],
)


_vector_mesh = plsc.VectorSubcoreMesh(core_axis_name="c", subcore_axis_name="s")

_sc_params = pltpu.CompilerParams()
if "needs_layout_passes" in pltpu.CompilerParams.__dataclass_fields__:
    _sc_params = dataclasses.replace(_sc_params, needs_layout_passes=False)


@functools.partial(
    pl.kernel,
    compiler_params=_sc_params,
    out_type=(
        jax.ShapeDtypeStruct((E,), jnp.float32),
        jax.ShapeDtypeStruct((NC, NPAD), jnp.float32),
    ),
    mesh=_vector_mesh,
    scratch_types=[
        pltpu.VMEM((NPAD,), jnp.float32),
        pltpu.VMEM((NPAD,), jnp.float32),
        pltpu.VMEM((16,), jnp.float32),
        pltpu.VMEM((KW,), jnp.int32),
        pltpu.VMEM((KW,), jnp.int32),
        pltpu.VMEM((KW,), jnp.float32),
        pltpu.VMEM((NPAD,), jnp.float32),
        pltpu.VMEM((RSUB,), jnp.float32),
        pltpu.VMEM((RSUB,), jnp.float32),
        pltpu.VMEM_SHARED((NS, NPAD), jnp.float32),
    ],
)
def _edge_w(asrc_hbm, adst_hbm, negm_hbm, src_hbm, dst_hbm, w_hbm, s_hbm,
            asrc_v, adst_v, negm_v, src_v, dst_v, w_v, sacc_v, sred_v, stmp_v,
            s_st):
    cid = lax.axis_index("c")
    sid = lax.axis_index("s")
    wid = sid * NC + cid

    pltpu.sync_copy(asrc_hbm, asrc_v)
    pltpu.sync_copy(adst_hbm, adst_v)
    pltpu.sync_copy(negm_hbm, negm_v)
    negm16 = negm_v[...]
    zf = jnp.zeros((16,), jnp.float32)
    iota16 = lax.iota(jnp.int32, 16)

    @pl.loop(0, NPAD, step=16)
    def _(t):
        sacc_v[pl.ds(t, 16)] = zf

    ebase = wid * EW

    @pl.loop(0, EW // KW)
    def _(c):
        base = ebase + c * KW
        pltpu.sync_copy(src_hbm.at[pl.ds(base, KW)], src_v)
        pltpu.sync_copy(dst_hbm.at[pl.ds(base, KW)], dst_v)

        @pl.loop(0, KW, step=16)
        def _(g):
            s16 = src_v[pl.ds(g, 16)]
            d16 = dst_v[pl.ds(g, 16)]
            av = plsc.load_gather(asrc_v, [s16])
            bv = plsc.load_gather(adst_v, [d16])
            z = av + bv
            alpha = jnp.maximum(z, 0.2 * z)
            w = jnp.exp(alpha + negm16)
            w_v[pl.ds(g, 16)] = w
            for j in range(16):
                plsc.addupdate_scatter(sacc_v, [d16], w, mask=iota16 == j)

        pltpu.sync_copy(w_v, w_hbm.at[pl.ds(base, KW)])

    pltpu.sync_copy(sacc_v, s_st.at[sid])
    plsc.subcore_barrier()

    row0 = sid * RSUB

    pltpu.sync_copy(s_st.at[0, pl.ds(row0, RSUB)], sred_v)
    for k in range(1, NS):
        pltpu.sync_copy(s_st.at[k, pl.ds(row0, RSUB)], stmp_v)

        @pl.loop(0, RSUB, step=16)
        def _(t):
            sred_v[pl.ds(t, 16)] = sred_v[pl.ds(t, 16)] + stmp_v[pl.ds(t, 16)]

    pltpu.sync_copy(sred_v, s_hbm.at[cid, pl.ds(row0, RSUB)])


NBUF = 3
NIB = 4


@functools.partial(
    pl.kernel,
    compiler_params=_sc_params,
    out_type=jax.ShapeDtypeStruct((NC, NACC, C), jnp.float32),
    mesh=_vector_mesh,
    scratch_types=[
        [pltpu.VMEM((K,), jnp.int32)] * NIB,
        [pltpu.VMEM((K,), jnp.int32)] * NIB,
        [pltpu.VMEM((K,), jnp.float32)] * NIB,
        [pltpu.VMEM((K, C), jnp.float32)] * NBUF,
        pltpu.VMEM_SHARED((NACC, C), jnp.float32),
        [pltpu.SemaphoreType.DMA] * NBUF,
        [pltpu.SemaphoreType.DMA] * NBUF,
        [pltpu.SemaphoreType.DMA] * NIB,
    ],
)
def _gat_scatter(xw_hbm, w_hbm, src_hbm, dst_hbm, o_part,
                 src_v, dst_v, w_v, rows_v, out_sh, gsem, ssem, isem):
    cid = lax.axis_index("c")
    sid = lax.axis_index("s")
    wid = sid * NC + cid

    zf = jnp.zeros((16,), jnp.float32)

    @pl.loop(0, K)
    def _(r):
        for g in range(NG):
            rows_v[0][r, pl.ds(g * 16, 16)] = zf

    row0 = sid * RSUB
    for j in range(RSUB // K):
        pltpu.sync_copy(rows_v[0], out_sh.at[pl.ds(row0 + j * K, K)])

    plsc.subcore_barrier()

    ebase = wid * EW
    zi = jnp.zeros((16,), jnp.int32)

    def load_idx_sync(c, i4):
        base = ebase + c * K
        pltpu.sync_copy(src_hbm.at[pl.ds(base, K)], src_v[i4])
        pltpu.sync_copy(dst_hbm.at[pl.ds(base, K)], dst_v[i4])
        pltpu.sync_copy(w_hbm.at[pl.ds(base, K)], w_v[i4])

    def load_idx_async(c, i4):
        base = ebase + c * K
        pltpu.async_copy(src_hbm.at[pl.ds(base, K)], src_v[i4], isem[i4])
        pltpu.async_copy(dst_hbm.at[pl.ds(base, K)], dst_v[i4], isem[i4])
        pltpu.async_copy(w_hbm.at[pl.ds(base, K)], w_v[i4], isem[i4])

    def wait_idx(c, i4):
        base = ebase + c * K
        pltpu.make_async_copy(src_hbm.at[pl.ds(base, K)], src_v[i4], isem[i4]).wait()
        pltpu.make_async_copy(dst_hbm.at[pl.ds(base, K)], dst_v[i4], isem[i4]).wait()
        pltpu.make_async_copy(w_hbm.at[pl.ds(base, K)], w_v[i4], isem[i4]).wait()

    def start_gather(i4, b):
        pltpu.async_copy(xw_hbm.at[src_v[i4]], rows_v[b], gsem[b])

    def wait_gather(i4, b):
        pltpu.make_async_copy(xw_hbm.at[src_v[i4]], rows_v[b], gsem[b]).wait()

    def scale_and_scatter(i4, b):
        @pl.loop(0, K, unroll=4)
        def _(r):
            wv = plsc.load_gather(w_v[i4], [zi + r])
            for g2 in range(NG):
                rows_v[b][r, pl.ds(g2 * 16, 16)] = (
                    rows_v[b][r, pl.ds(g2 * 16, 16)] * wv)

        pltpu.async_copy(rows_v[b], out_sh.at[dst_v[i4]], ssem[b], add=True)

    def wait_scatter(i4, b):
        pltpu.make_async_copy(rows_v[b], out_sh.at[dst_v[i4]], ssem[b]).wait()

    load_idx_sync(0, 0)
    start_gather(0, 0)
    load_idx_sync(1, 1)
    start_gather(1, 1)
    load_idx_async(2, 2)
    wait_gather(0, 0)
    scale_and_scatter(0, 0)
    wait_idx(2, 2)
    start_gather(2, 2)
    load_idx_async(3, 3)
    wait_gather(1, 1)
    scale_and_scatter(1, 1)

    @pl.loop(0, (NCHUNK - 5) // 12)
    def _(i):
        for u in range(12):
            c = 12 * i + 2 + u
            b = (2 + u) % NBUF
            bn = (b + 1) % NBUF
            i4 = (2 + u) % NIB
            i4n = (i4 + 1) % NIB
            i4nn = (i4 + 2) % NIB
            wait_scatter(i4nn, bn)
            wait_idx(c + 1, i4n)
            start_gather(i4n, bn)
            base2 = ebase + (c + 2) * K
            pltpu.async_copy(src_hbm.at[pl.ds(base2, K)], src_v[i4nn], isem[i4nn])
            pltpu.async_copy(dst_hbm.at[pl.ds(base2, K)], dst_v[i4nn], isem[i4nn])
            pltpu.async_copy(w_hbm.at[pl.ds(base2, K)], w_v[i4nn], isem[i4nn])
            wait_gather(i4, b)
            scale_and_scatter(i4, b)

    for c in range(NCHUNK - 3, NCHUNK):
        b = c % NBUF
        bn = (b + 1) % NBUF
        i4 = c % NIB
        i4n = (i4 + 1) % NIB
        i4nn = (i4 + 2) % NIB
        wait_scatter(i4nn, bn)
        if c + 1 < NCHUNK:
            wait_idx(c + 1, i4n)
            start_gather(i4n, bn)
        if c + 2 < NCHUNK:
            load_idx_async(c + 2, i4nn)
        wait_gather(i4, b)
        scale_and_scatter(i4, b)

    wait_scatter((NCHUNK - 2) % NIB, (NCHUNK - 2) % NBUF)
    wait_scatter((NCHUNK - 1) % NIB, (NCHUNK - 1) % NBUF)

    plsc.subcore_barrier()

    pltpu.sync_copy(out_sh.at[pl.ds(row0, RSUB)], o_part.at[cid, pl.ds(row0, RSUB)])


BLKC = 1000


def _final_body(o_ref, s_ref, b_ref, fw_ref, fb_ref, y_ref):
    o = o_ref[0] + o_ref[1]
    s = s_ref[0] + s_ref[1]
    h = o / (s + 1e-16) + b_ref[...]
    h = jnp.maximum(h, 0.0)
    y = jnp.sum(h * fw_ref[...], axis=1, keepdims=True) + fb_ref[0]
    y_ref[...] = jax.nn.sigmoid(y)


_final_call = pl.pallas_call(
    _final_body,
    grid=(N // BLKC,),
    in_specs=[
        pl.BlockSpec((NC, BLKC, C), lambda i: (0, i, 0)),
        pl.BlockSpec((NC, BLKC, 1), lambda i: (0, i, 0)),
        pl.BlockSpec((1, C), lambda i: (0, 0)),
        pl.BlockSpec((1, C), lambda i: (0, 0)),
        pl.BlockSpec(memory_space=pltpu.SMEM),
    ],
    out_specs=pl.BlockSpec((BLKC, 1), lambda i: (i, 0)),
    out_shape=jax.ShapeDtypeStruct((N, 1), jnp.float32),
)


def kernel(x, edge_index, W, att_src, att_dst, bias, fc_W, fc_b):
    src = edge_index[0]
    dst = edge_index[1]
    x_pad = jnp.pad(x, ((0, NPAD - N), (0, 0)))
    att_s = att_src.reshape(1, C)
    att_d = att_dst.reshape(1, C)
    xw, asrc2, adst2, negm = _prep_call(x_pad, W, att_s, att_d)
    asrc = asrc2.reshape(NPAD)
    adst = adst2.reshape(NPAD)
    negm16 = jnp.broadcast_to(negm.reshape(1), (16,))
    w_e, s_part = _edge_w(asrc, adst, negm16, src, dst)
    o_part = _gat_scatter(xw, w_e, src, dst)
    s_part3 = s_part.reshape(NC, NPAD, 1)
    return _final_call(o_part, s_part3, bias.reshape(1, C), fc_W.reshape(1, C),
                       fc_b)
